# baseline (speedup 1.0000x reference)
import functools
import math

import jax
import jax.numpy as jnp
from jax import lax
from jax.experimental import pallas as pl
from jax.experimental.pallas import tpu as pltpu
from jax.experimental.pallas import tpu_sc as plsc

NN = 10000
NP = 10240
EE = 320000
EPS = 1e-5

NC = 2
NS = 16
RPT = NP // NS
EPT = EE // NS
CH = 80
BM = 512


def _sc_mesh():
    return plsc.VectorSubcoreMesh(core_axis_name="c", subcore_axis_name="s")


def _degree(dst, zeros_init, ones_rows):
    @functools.partial(
        pl.kernel,
        out_type=jax.ShapeDtypeStruct((NC, NP, 128), jnp.float32),
        mesh=_sc_mesh(),
        scratch_types=[
            pltpu.VMEM_SHARED((NP, 128), jnp.float32),
            pltpu.VMEM((CH,), jnp.int32),
            pltpu.VMEM((CH, 128), jnp.float32),
        ],
    )
    def deg_kernel(dst_hbm, zinit_hbm, ones_hbm, out_hbm, acc, didx, ones_v):
        c = lax.axis_index("c")
        s = lax.axis_index("s")
        pltpu.sync_copy(ones_hbm, ones_v)
        pltpu.sync_copy(zinit_hbm.at[pl.ds(s * RPT, RPT)],
                        acc.at[pl.ds(s * RPT, RPT)])
        plsc.subcore_barrier()
        epc = EE // NC
        ept = epc // NS
        base = c * epc + s * ept

        def eloop(j, _):
            pltpu.sync_copy(dst_hbm.at[pl.ds(base + j * CH, CH)], didx)
            pltpu.sync_copy(ones_v, acc.at[didx], add=True)
            return 0

        lax.fori_loop(0, ept // CH, eloop, 0)
        plsc.subcore_barrier()
        pltpu.sync_copy(acc.at[pl.ds(s * RPT, RPT)],
                        out_hbm.at[c, pl.ds(s * RPT, RPT)])

    return deg_kernel(dst, zeros_init, ones_rows)


NITER = EPT // CH
NBUF = 2


@functools.lru_cache(maxsize=None)
def _make_agg(NB, C):
    NBC = NB // NC
    RS = 4
    IS = 8
    NMAIN = (NITER // IS) * IS

    @functools.partial(
        pl.kernel,
        out_type=jax.ShapeDtypeStruct((NB, NP, C), jnp.float32),
        mesh=_sc_mesh(),
        scratch_types=[
            pltpu.VMEM_SHARED((NP, C), jnp.float32),
            pltpu.VMEM((IS, CH), jnp.int32),
            pltpu.VMEM((IS, CH), jnp.int32),
            pltpu.VMEM((RS, CH, C), jnp.float32),
            pltpu.SemaphoreType.DMA,
            pltpu.SemaphoreType.DMA,
            pltpu.SemaphoreType.DMA,
        ],
    )
    def agg_kernel(y_hbm, src_hbm, dst_hbm, out_hbm, acc, sidx, didx, rows,
                   isem, gsem, ssem):
        c = lax.axis_index("c")
        s = lax.axis_index("s")
        ebase = s * EPT

        def fetch_idx(j, sl):
            pltpu.async_copy(src_hbm.at[pl.ds(ebase + j * CH, CH)],
                             sidx.at[sl], isem)
            pltpu.async_copy(dst_hbm.at[pl.ds(ebase + j * CH, CH)],
                             didx.at[sl], isem)

        def wait_idx(sl):
            pltpu.make_async_copy(src_hbm.at[pl.ds(0, CH)], sidx.at[sl],
                                  isem).wait()
            pltpu.make_async_copy(dst_hbm.at[pl.ds(0, CH)], didx.at[sl],
                                  isem).wait()

        def wait_gather(y_fb, rb):
            pltpu.make_async_copy(y_fb.at[sidx.at[0]], rows.at[rb],
                                  gsem).wait()

        def wait_scatter(rb):
            pltpu.make_async_copy(rows.at[rb], acc.at[didx.at[0]],
                                  ssem).wait()

        for bi in range(NBC):
            fb = c * NBC + bi
            y_fb = y_hbm.at[fb]
            pltpu.sync_copy(y_hbm.at[fb, pl.ds(s * RPT, RPT)],
                            acc.at[pl.ds(s * RPT, RPT)])
            plsc.subcore_barrier()

            for j0 in range(5):
                fetch_idx(j0, j0)
            wait_idx(0)
            pltpu.async_copy(y_fb.at[sidx.at[0]], rows.at[0], gsem)

            def step(j, ib, rb):
                wait_gather(y_fb, rb)
                pltpu.async_copy(rows.at[rb], acc.at[didx.at[ib]], ssem,
                                 add=True)

                @pl.when(j >= 3)
                def _():
                    wait_scatter((rb + 1) % RS)

                @pl.when(j + 1 < NITER)
                def _():
                    wait_idx((ib + 1) % IS)
                    pltpu.async_copy(y_fb.at[sidx.at[(ib + 1) % IS]],
                                     rows.at[(rb + 1) % RS], gsem)

                @pl.when(j + 5 < NITER)
                def _():
                    fetch_idx(j + 5, (ib + 5) % IS)

            def group(g, _):
                for b in range(IS):
                    step(g * IS + b, b, b % RS)
                return 0

            lax.fori_loop(0, NMAIN // IS, group, 0)
            for j in range(NMAIN, NITER):
                step(j, j % IS, j % RS)
            for j in range(NITER - 3, NITER):
                wait_scatter(j % RS)
            plsc.subcore_barrier()
            pltpu.sync_copy(acc.at[pl.ds(s * RPT, RPT)],
                            out_hbm.at[fb, pl.ds(s * RPT, RPT)])
            if bi + 1 < NBC:
                plsc.subcore_barrier()

    return agg_kernel


def _mm_first(x, W, deg2):
    F = W.shape[1]

    def body(x_ref, w_ref, deg_ref, y_ref, dinv_ref):
        deg = deg_ref[0] + deg_ref[1] + 1.0
        dinv = lax.rsqrt(deg)
        dinv_ref[...] = dinv
        d1 = dinv[:, :1]
        y = jnp.dot(x_ref[...], w_ref[...],
                    preferred_element_type=jnp.float32) * d1
        for fbl in range(F // 128):
            y_ref[fbl] = y[:, fbl * 128:(fbl + 1) * 128]

    return pl.pallas_call(
        body,
        grid=(NP // BM,),
        in_specs=[
            pl.BlockSpec((BM, x.shape[1]), lambda i: (i, 0)),
            pl.BlockSpec(W.shape, lambda i: (0, 0)),
            pl.BlockSpec((NC, BM, 128), lambda i: (0, i, 0)),
        ],
        out_specs=[
            pl.BlockSpec((F // 128, BM, 128), lambda i: (0, i, 0)),
            pl.BlockSpec((BM, 128), lambda i: (i, 0)),
        ],
        out_shape=[
            jax.ShapeDtypeStruct((F // 128, NP, 128), jnp.float32),
            jax.ShapeDtypeStruct((NP, 128), jnp.float32),
        ],
    )(x, W, deg2)


def _mm_mid(agg, dinv16, b, g, be, W, C_out):
    NBin, _, C_in = agg.shape
    Fout = W.shape[1]
    NBo = Fout // C_out
    bnscale = 1.0 / math.sqrt(1.0 + EPS)

    def body(a_ref, dinv_ref, b_ref, g_ref, be_ref, w_ref, y_ref):
        d1 = dinv_ref[:, :1]
        acc = jnp.zeros((BM, Fout), jnp.float32)
        for kb in range(NBin):
            sl = slice(kb * C_in, (kb + 1) * C_in)
            h = a_ref[kb] * d1 + b_ref[0:1, sl]
            h = jnp.where(h > 0, h, jnp.exp(h) - 1.0)
            h = h * (g_ref[0:1, sl] * bnscale) + be_ref[0:1, sl]
            acc = acc + jnp.dot(h, w_ref[sl, :],
                                preferred_element_type=jnp.float32)
        y = acc * d1
        for fbl in range(NBo):
            y_ref[fbl] = y[:, fbl * C_out:(fbl + 1) * C_out]

    Fin = NBin * C_in
    return pl.pallas_call(
        body,
        grid=(NP // BM,),
        in_specs=[
            pl.BlockSpec((NBin, BM, C_in), lambda i: (0, i, 0)),
            pl.BlockSpec((BM, 128), lambda i: (i, 0)),
            pl.BlockSpec((1, Fin), lambda i: (0, 0)),
            pl.BlockSpec((1, Fin), lambda i: (0, 0)),
            pl.BlockSpec((1, Fin), lambda i: (0, 0)),
            pl.BlockSpec((Fin, Fout), lambda i: (0, 0)),
        ],
        out_specs=[pl.BlockSpec((NBo, BM, C_out), lambda i: (0, i, 0))],
        out_shape=[jax.ShapeDtypeStruct((NBo, NP, C_out), jnp.float32)],
    )(agg, dinv16, b, g, be, W)[0]


def _final(agg4, dinv16, b4):
    C = agg4.shape[2]
    NBin = agg4.shape[0]
    F = NBin * C

    def body(a_ref, dinv_ref, b_ref, o_ref):
        d1 = dinv_ref[:, :1]
        h = jnp.concatenate([a_ref[kb] for kb in range(NBin)], axis=1)
        o_ref[...] = h * d1 + b_ref[...]

    return pl.pallas_call(
        body,
        grid=(NP // BM,),
        in_specs=[
            pl.BlockSpec((NBin, BM, C), lambda i: (0, i, 0)),
            pl.BlockSpec((BM, 128), lambda i: (i, 0)),
            pl.BlockSpec((1, F), lambda i: (0, 0)),
        ],
        out_specs=pl.BlockSpec((BM, F), lambda i: (i, 0)),
        out_shape=jax.ShapeDtypeStruct((NP, F), jnp.float32),
    )(agg4, dinv16, b4)


def _padc(v, n):
    return jnp.pad(v, (0, n - v.shape[0]))[None, :]


def kernel(x, edge_index, W1, b1, W2, b2, W3, b3, W4, b4, g1, be1, g2, be2, g3, be3):
    src = edge_index[0]
    dst = edge_index[1]
    x = jnp.pad(x, ((0, NP - NN), (0, 0)))

    W1p = jnp.pad(W1, ((0, 0), (0, 512 - 500)))
    W2p = jnp.pad(W2, ((0, 512 - 500), (0, 1024 - 1000)))
    W3p = jnp.pad(W3, ((0, 1024 - 1000), (0, 512 - 500)))
    W4p = jnp.pad(W4, ((0, 512 - 500), (0, 256 - 50)))
    b1p, g1p, be1p = _padc(b1, 512), _padc(g1, 512), _padc(be1, 512)
    b2p, g2p, be2p = _padc(b2, 1024), _padc(g2, 1024), _padc(be2, 1024)
    b3p, g3p, be3p = _padc(b3, 512), _padc(g3, 512), _padc(be3, 512)
    b4p = _padc(b4, 256)

    zinit = jnp.zeros((NP, 128), jnp.float32)
    ones_rows = jnp.ones((CH, 128), jnp.float32)

    deg2 = _degree(dst, zinit, ones_rows)
    y1, dinv16 = _mm_first(x, W1p, deg2)
    a1 = _make_agg(4, 128)(y1, src, dst)
    y2 = _mm_mid(a1, dinv16, b1p, g1p, be1p, W2p, 128)
    a2 = _make_agg(8, 128)(y2, src, dst)
    y3 = _mm_mid(a2, dinv16, b2p, g2p, be2p, W3p, 128)
    a3 = _make_agg(4, 128)(y3, src, dst)
    y4 = _mm_mid(a3, dinv16, b3p, g3p, be3p, W4p, 128)
    a4 = _make_agg(2, 128)(y4, src, dst)
    out = _final(a4, dinv16, b4p)
    return out[:NN, :50]


# speedup vs baseline: 1.4873x; 1.4873x over previous
import functools
import math

import jax
import jax.numpy as jnp
from jax import lax
from jax.experimental import pallas as pl
from jax.experimental.pallas import tpu as pltpu
from jax.experimental.pallas import tpu_sc as plsc

NN = 10000
NP = 10240
EE = 320000
EPS = 1e-5

NC = 2
NS = 16
RPT = NP // NS
EPT = EE // NS
CH = 80
BM = 512


def _sc_mesh():
    return plsc.VectorSubcoreMesh(core_axis_name="c", subcore_axis_name="s")


def _degree(dst, zeros_init, ones_rows):
    @functools.partial(
        pl.kernel,
        out_type=jax.ShapeDtypeStruct((NC, NP, 128), jnp.float32),
        mesh=_sc_mesh(),
        scratch_types=[
            pltpu.VMEM_SHARED((NP, 128), jnp.float32),
            pltpu.VMEM((CH,), jnp.int32),
            pltpu.VMEM((CH, 128), jnp.float32),
        ],
    )
    def deg_kernel(dst_hbm, zinit_hbm, ones_hbm, out_hbm, acc, didx, ones_v):
        c = lax.axis_index("c")
        s = lax.axis_index("s")
        pltpu.sync_copy(ones_hbm, ones_v)
        pltpu.sync_copy(zinit_hbm.at[pl.ds(s * RPT, RPT)],
                        acc.at[pl.ds(s * RPT, RPT)])
        plsc.subcore_barrier()
        epc = EE // NC
        ept = epc // NS
        base = c * epc + s * ept

        def eloop(j, _):
            pltpu.sync_copy(dst_hbm.at[pl.ds(base + j * CH, CH)], didx)
            pltpu.sync_copy(ones_v, acc.at[didx], add=True)
            return 0

        lax.fori_loop(0, ept // CH, eloop, 0)
        plsc.subcore_barrier()
        pltpu.sync_copy(acc.at[pl.ds(s * RPT, RPT)],
                        out_hbm.at[c, pl.ds(s * RPT, RPT)])

    return deg_kernel(dst, zeros_init, ones_rows)


NITER = EPT // CH
NBUF = 2


@functools.lru_cache(maxsize=None)
def _make_agg(NB, C):
    NBC = NB // NC
    RS = 4
    IS = 8
    NMAIN = (NITER // IS) * IS

    @functools.partial(
        pl.kernel,
        out_type=jax.ShapeDtypeStruct((NB, NP, C), jnp.float32),
        mesh=_sc_mesh(),
        scratch_types=[
            pltpu.VMEM_SHARED((NP, C), jnp.float32),
            pltpu.VMEM((IS, CH), jnp.int32),
            pltpu.VMEM((IS, CH), jnp.int32),
            pltpu.VMEM((RS, CH, C), jnp.float32),
            pltpu.SemaphoreType.DMA,
            pltpu.SemaphoreType.DMA,
            pltpu.SemaphoreType.DMA,
        ],
    )
    def agg_kernel(y_hbm, src_hbm, dst_hbm, out_hbm, acc, sidx, didx, rows,
                   isem, gsem, ssem):
        c = lax.axis_index("c")
        s = lax.axis_index("s")
        ebase = s * EPT

        def fetch_idx(j, sl):
            pltpu.async_copy(src_hbm.at[pl.ds(ebase + j * CH, CH)],
                             sidx.at[sl], isem)
            pltpu.async_copy(dst_hbm.at[pl.ds(ebase + j * CH, CH)],
                             didx.at[sl], isem)

        def wait_idx(sl):
            pltpu.make_async_copy(src_hbm.at[pl.ds(0, CH)], sidx.at[sl],
                                  isem).wait()
            pltpu.make_async_copy(dst_hbm.at[pl.ds(0, CH)], didx.at[sl],
                                  isem).wait()

        def wait_gather(y_fb, rb):
            pltpu.make_async_copy(y_fb.at[sidx.at[0]], rows.at[rb],
                                  gsem).wait()

        def wait_scatter(rb):
            pltpu.make_async_copy(rows.at[rb], acc.at[didx.at[0]],
                                  ssem).wait()

        for bi in range(NBC):
            fb = c * NBC + bi
            y_fb = y_hbm.at[fb]
            pltpu.sync_copy(y_hbm.at[fb, pl.ds(s * RPT, RPT)],
                            acc.at[pl.ds(s * RPT, RPT)])
            plsc.subcore_barrier()

            for j0 in range(5):
                fetch_idx(j0, j0)
            wait_idx(0)
            pltpu.async_copy(y_fb.at[sidx.at[0]], rows.at[0], gsem)
            wait_idx(1)
            pltpu.async_copy(y_fb.at[sidx.at[1]], rows.at[1], gsem)

            def step(j, ib, rb):
                wait_gather(y_fb, rb)
                pltpu.async_copy(rows.at[rb], acc.at[didx.at[ib]], ssem,
                                 add=True)

                @pl.when(j >= 2)
                def _():
                    wait_scatter((rb + 2) % RS)

                @pl.when(j + 2 < NITER)
                def _():
                    wait_idx((ib + 2) % IS)
                    pltpu.async_copy(y_fb.at[sidx.at[(ib + 2) % IS]],
                                     rows.at[(rb + 2) % RS], gsem)

                @pl.when(j + 5 < NITER)
                def _():
                    fetch_idx(j + 5, (ib + 5) % IS)

            def group(g, _):
                for b in range(IS):
                    step(g * IS + b, b, b % RS)
                return 0

            lax.fori_loop(0, NMAIN // IS, group, 0)
            for j in range(NMAIN, NITER):
                step(j, j % IS, j % RS)
            for j in range(NITER - 2, NITER):
                wait_scatter(j % RS)
            plsc.subcore_barrier()
            pltpu.sync_copy(acc.at[pl.ds(s * RPT, RPT)],
                            out_hbm.at[fb, pl.ds(s * RPT, RPT)])
            if bi + 1 < NBC:
                plsc.subcore_barrier()

    return agg_kernel


def _mm_first(x, W, deg2):
    F = W.shape[1]

    def body(x_ref, w_ref, deg_ref, y_ref, dinv_ref):
        deg = deg_ref[0] + deg_ref[1] + 1.0
        dinv = lax.rsqrt(deg)
        dinv_ref[...] = dinv
        d1 = dinv[:, :1]
        y = jnp.dot(x_ref[...], w_ref[...],
                    preferred_element_type=jnp.float32) * d1
        for fbl in range(F // 128):
            y_ref[fbl] = y[:, fbl * 128:(fbl + 1) * 128]

    return pl.pallas_call(
        body,
        grid=(NP // BM,),
        in_specs=[
            pl.BlockSpec((BM, x.shape[1]), lambda i: (i, 0)),
            pl.BlockSpec(W.shape, lambda i: (0, 0)),
            pl.BlockSpec((NC, BM, 128), lambda i: (0, i, 0)),
        ],
        out_specs=[
            pl.BlockSpec((F // 128, BM, 128), lambda i: (0, i, 0)),
            pl.BlockSpec((BM, 128), lambda i: (i, 0)),
        ],
        out_shape=[
            jax.ShapeDtypeStruct((F // 128, NP, 128), jnp.float32),
            jax.ShapeDtypeStruct((NP, 128), jnp.float32),
        ],
    )(x, W, deg2)


def _mm_mid(agg, dinv16, b, g, be, W, C_out):
    NBin, _, C_in = agg.shape
    Fout = W.shape[1]
    NBo = Fout // C_out
    bnscale = 1.0 / math.sqrt(1.0 + EPS)

    def body(a_ref, dinv_ref, b_ref, g_ref, be_ref, w_ref, y_ref):
        d1 = dinv_ref[:, :1]
        acc = jnp.zeros((BM, Fout), jnp.float32)
        for kb in range(NBin):
            sl = slice(kb * C_in, (kb + 1) * C_in)
            h = a_ref[kb] * d1 + b_ref[0:1, sl]
            h = jnp.where(h > 0, h, jnp.exp(h) - 1.0)
            h = h * (g_ref[0:1, sl] * bnscale) + be_ref[0:1, sl]
            acc = acc + jnp.dot(h, w_ref[sl, :],
                                preferred_element_type=jnp.float32)
        y = acc * d1
        for fbl in range(NBo):
            y_ref[fbl] = y[:, fbl * C_out:(fbl + 1) * C_out]

    Fin = NBin * C_in
    return pl.pallas_call(
        body,
        grid=(NP // BM,),
        in_specs=[
            pl.BlockSpec((NBin, BM, C_in), lambda i: (0, i, 0)),
            pl.BlockSpec((BM, 128), lambda i: (i, 0)),
            pl.BlockSpec((1, Fin), lambda i: (0, 0)),
            pl.BlockSpec((1, Fin), lambda i: (0, 0)),
            pl.BlockSpec((1, Fin), lambda i: (0, 0)),
            pl.BlockSpec((Fin, Fout), lambda i: (0, 0)),
        ],
        out_specs=[pl.BlockSpec((NBo, BM, C_out), lambda i: (0, i, 0))],
        out_shape=[jax.ShapeDtypeStruct((NBo, NP, C_out), jnp.float32)],
    )(agg, dinv16, b, g, be, W)[0]


def _final(agg4, dinv16, b4):
    C = agg4.shape[2]
    NBin = agg4.shape[0]
    F = NBin * C

    def body(a_ref, dinv_ref, b_ref, o_ref):
        d1 = dinv_ref[:, :1]
        h = jnp.concatenate([a_ref[kb] for kb in range(NBin)], axis=1)
        o_ref[...] = h * d1 + b_ref[...]

    return pl.pallas_call(
        body,
        grid=(NP // BM,),
        in_specs=[
            pl.BlockSpec((NBin, BM, C), lambda i: (0, i, 0)),
            pl.BlockSpec((BM, 128), lambda i: (i, 0)),
            pl.BlockSpec((1, F), lambda i: (0, 0)),
        ],
        out_specs=pl.BlockSpec((BM, F), lambda i: (i, 0)),
        out_shape=jax.ShapeDtypeStruct((NP, F), jnp.float32),
    )(agg4, dinv16, b4)


def _padc(v, n):
    return jnp.pad(v, (0, n - v.shape[0]))[None, :]


def kernel(x, edge_index, W1, b1, W2, b2, W3, b3, W4, b4, g1, be1, g2, be2, g3, be3):
    src = edge_index[0]
    dst = edge_index[1]
    x = jnp.pad(x, ((0, NP - NN), (0, 0)))

    W1p = jnp.pad(W1, ((0, 0), (0, 512 - 500)))
    W2p = jnp.pad(W2, ((0, 512 - 500), (0, 1024 - 1000)))
    W3p = jnp.pad(W3, ((0, 1024 - 1000), (0, 512 - 500)))
    W4p = jnp.pad(W4, ((0, 512 - 500), (0, 256 - 50)))
    b1p, g1p, be1p = _padc(b1, 512), _padc(g1, 512), _padc(be1, 512)
    b2p, g2p, be2p = _padc(b2, 1024), _padc(g2, 1024), _padc(be2, 1024)
    b3p, g3p, be3p = _padc(b3, 512), _padc(g3, 512), _padc(be3, 512)
    b4p = _padc(b4, 256)

    zinit = jnp.zeros((NP, 128), jnp.float32)
    ones_rows = jnp.ones((CH, 128), jnp.float32)

    deg2 = _degree(dst, zinit, ones_rows)
    y1, dinv16 = _mm_first(x, W1p, deg2)
    a1 = _make_agg(4, 128)(y1, src, dst)
    y2 = _mm_mid(a1, dinv16, b1p, g1p, be1p, W2p, 128)
    a2 = _make_agg(8, 128)(y2, src, dst)
    y3 = _mm_mid(a2, dinv16, b2p, g2p, be2p, W3p, 128)
    a3 = _make_agg(4, 128)(y3, src, dst)
    y4 = _mm_mid(a3, dinv16, b3p, g3p, be3p, W4p, 128)
    a4 = _make_agg(2, 128)(y4, src, dst)
    out = _final(a4, dinv16, b4p)
    return out[:NN, :50]


# speedup vs baseline: 1.5538x; 1.0447x over previous
import functools
import math

import jax
import jax.numpy as jnp
from jax import lax
from jax.experimental import pallas as pl
from jax.experimental.pallas import tpu as pltpu
from jax.experimental.pallas import tpu_sc as plsc

NN = 10000
NP = 10240
EE = 320000
EPS = 1e-5

NC = 2
NS = 16
RPT = NP // NS
EPT = EE // NS
CH = 80
BM = 512


def _sc_mesh():
    return plsc.VectorSubcoreMesh(core_axis_name="c", subcore_axis_name="s")


def _degree(dst, zeros_init, ones_rows):
    @functools.partial(
        pl.kernel,
        out_type=jax.ShapeDtypeStruct((NC, NP, 128), jnp.float32),
        mesh=_sc_mesh(),
        scratch_types=[
            pltpu.VMEM_SHARED((NP, 128), jnp.float32),
            pltpu.VMEM((CH,), jnp.int32),
            pltpu.VMEM((CH, 128), jnp.float32),
        ],
    )
    def deg_kernel(dst_hbm, zinit_hbm, ones_hbm, out_hbm, acc, didx, ones_v):
        c = lax.axis_index("c")
        s = lax.axis_index("s")
        pltpu.sync_copy(ones_hbm, ones_v)
        pltpu.sync_copy(zinit_hbm.at[pl.ds(s * RPT, RPT)],
                        acc.at[pl.ds(s * RPT, RPT)])
        plsc.subcore_barrier()
        epc = EE // NC
        ept = epc // NS
        base = c * epc + s * ept

        def eloop(j, _):
            pltpu.sync_copy(dst_hbm.at[pl.ds(base + j * CH, CH)], didx)
            pltpu.sync_copy(ones_v, acc.at[didx], add=True)
            return 0

        lax.fori_loop(0, ept // CH, eloop, 0)
        plsc.subcore_barrier()
        pltpu.sync_copy(acc.at[pl.ds(s * RPT, RPT)],
                        out_hbm.at[c, pl.ds(s * RPT, RPT)])

    return deg_kernel(dst, zeros_init, ones_rows)


NITER = EPT // CH
NBUF = 2


@functools.lru_cache(maxsize=None)
def _make_agg(NB, C, edge_split=False):
    NBC = 1 if edge_split else NB // NC
    RS = 4
    IS = 8
    NITERL = (EPT // NC if edge_split else EPT) // CH
    NMAIN = (NITERL // IS) * IS
    NOUT = NC if edge_split else NB

    @functools.partial(
        pl.kernel,
        out_type=jax.ShapeDtypeStruct((NOUT, NP, C), jnp.float32),
        mesh=_sc_mesh(),
        scratch_types=[
            pltpu.VMEM_SHARED((NP, C), jnp.float32),
            pltpu.VMEM((IS, CH), jnp.int32),
            pltpu.VMEM((IS, CH), jnp.int32),
            pltpu.VMEM((RS, CH, C), jnp.float32),
            pltpu.SemaphoreType.DMA,
            pltpu.SemaphoreType.DMA,
            pltpu.SemaphoreType.DMA,
        ],
    )
    def agg_kernel(y_hbm, src_hbm, dst_hbm, out_hbm, acc, sidx, didx, rows,
                   isem, gsem, ssem):
        c = lax.axis_index("c")
        s = lax.axis_index("s")
        if edge_split:
            ebase = c * (EE // NC) + s * (EPT // NC)
        else:
            ebase = s * EPT

        def fetch_idx(j, sl):
            pltpu.async_copy(src_hbm.at[pl.ds(ebase + j * CH, CH)],
                             sidx.at[sl], isem)
            pltpu.async_copy(dst_hbm.at[pl.ds(ebase + j * CH, CH)],
                             didx.at[sl], isem)

        def wait_idx(sl):
            pltpu.make_async_copy(src_hbm.at[pl.ds(0, CH)], sidx.at[sl],
                                  isem).wait()
            pltpu.make_async_copy(dst_hbm.at[pl.ds(0, CH)], didx.at[sl],
                                  isem).wait()

        def wait_gather(y_fb, rb):
            pltpu.make_async_copy(y_fb.at[sidx.at[0]], rows.at[rb],
                                  gsem).wait()

        def wait_scatter(rb):
            pltpu.make_async_copy(rows.at[rb], acc.at[didx.at[0]],
                                  ssem).wait()

        for bi in range(NBC):
            fb = 0 if edge_split else c * NBC + bi
            ob = c if edge_split else fb
            y_fb = y_hbm.at[fb]
            pltpu.sync_copy(y_hbm.at[fb, pl.ds(s * RPT, RPT)],
                            acc.at[pl.ds(s * RPT, RPT)])
            plsc.subcore_barrier()

            for j0 in range(5):
                fetch_idx(j0, j0)
            wait_idx(0)
            pltpu.async_copy(y_fb.at[sidx.at[0]], rows.at[0], gsem)
            wait_idx(1)
            pltpu.async_copy(y_fb.at[sidx.at[1]], rows.at[1], gsem)

            def step(j, ib, rb):
                wait_gather(y_fb, rb)
                pltpu.async_copy(rows.at[rb], acc.at[didx.at[ib]], ssem,
                                 add=True)

                @pl.when(j >= 2)
                def _():
                    wait_scatter((rb + 2) % RS)

                @pl.when(j + 2 < NITERL)
                def _():
                    wait_idx((ib + 2) % IS)
                    pltpu.async_copy(y_fb.at[sidx.at[(ib + 2) % IS]],
                                     rows.at[(rb + 2) % RS], gsem)

                @pl.when(j + 5 < NITERL)
                def _():
                    fetch_idx(j + 5, (ib + 5) % IS)

            def group(g, _):
                for b in range(IS):
                    step(g * IS + b, b, b % RS)
                return 0

            lax.fori_loop(0, NMAIN // IS, group, 0)
            for j in range(NMAIN, NITERL):
                step(j, j % IS, j % RS)
            for j in range(NITERL - 2, NITERL):
                wait_scatter(j % RS)
            plsc.subcore_barrier()
            pltpu.sync_copy(acc.at[pl.ds(s * RPT, RPT)],
                            out_hbm.at[ob, pl.ds(s * RPT, RPT)])
            if bi + 1 < NBC:
                plsc.subcore_barrier()

    return agg_kernel


def _mm_first(x, W, deg2):
    F = W.shape[1]

    def body(x_ref, w_ref, deg_ref, y_ref, dinv_ref):
        deg = deg_ref[0] + deg_ref[1] + 1.0
        dinv = lax.rsqrt(deg)
        dinv_ref[...] = dinv
        d1 = dinv[:, :1]
        y = jnp.dot(x_ref[...], w_ref[...],
                    preferred_element_type=jnp.float32) * d1
        for fbl in range(F // 128):
            y_ref[fbl] = y[:, fbl * 128:(fbl + 1) * 128]

    return pl.pallas_call(
        body,
        grid=(NP // BM,),
        in_specs=[
            pl.BlockSpec((BM, x.shape[1]), lambda i: (i, 0)),
            pl.BlockSpec(W.shape, lambda i: (0, 0)),
            pl.BlockSpec((NC, BM, 128), lambda i: (0, i, 0)),
        ],
        out_specs=[
            pl.BlockSpec((F // 128, BM, 128), lambda i: (0, i, 0)),
            pl.BlockSpec((BM, 128), lambda i: (i, 0)),
        ],
        out_shape=[
            jax.ShapeDtypeStruct((F // 128, NP, 128), jnp.float32),
            jax.ShapeDtypeStruct((NP, 128), jnp.float32),
        ],
    )(x, W, deg2)


def _mm_mid(agg, dinv16, b, g, be, W, C_out):
    NBin, _, C_in = agg.shape
    Fout = W.shape[1]
    NBo = Fout // C_out
    bnscale = 1.0 / math.sqrt(1.0 + EPS)

    def body(a_ref, dinv_ref, b_ref, g_ref, be_ref, w_ref, y_ref):
        d1 = dinv_ref[:, :1]
        acc = jnp.zeros((BM, Fout), jnp.float32)
        for kb in range(NBin):
            sl = slice(kb * C_in, (kb + 1) * C_in)
            h = a_ref[kb] * d1 + b_ref[0:1, sl]
            h = jnp.where(h > 0, h, jnp.exp(h) - 1.0)
            h = h * (g_ref[0:1, sl] * bnscale) + be_ref[0:1, sl]
            acc = acc + jnp.dot(h, w_ref[sl, :],
                                preferred_element_type=jnp.float32)
        y = acc * d1
        for fbl in range(NBo):
            y_ref[fbl] = y[:, fbl * C_out:(fbl + 1) * C_out]

    Fin = NBin * C_in
    return pl.pallas_call(
        body,
        grid=(NP // BM,),
        in_specs=[
            pl.BlockSpec((NBin, BM, C_in), lambda i: (0, i, 0)),
            pl.BlockSpec((BM, 128), lambda i: (i, 0)),
            pl.BlockSpec((1, Fin), lambda i: (0, 0)),
            pl.BlockSpec((1, Fin), lambda i: (0, 0)),
            pl.BlockSpec((1, Fin), lambda i: (0, 0)),
            pl.BlockSpec((Fin, Fout), lambda i: (0, 0)),
        ],
        out_specs=[pl.BlockSpec((NBo, BM, C_out), lambda i: (0, i, 0))],
        out_shape=[jax.ShapeDtypeStruct((NBo, NP, C_out), jnp.float32)],
    )(agg, dinv16, b, g, be, W)[0]


def _final(agg4, y4, dinv16, b4):
    C = agg4.shape[2]

    def body(a_ref, y_ref, dinv_ref, b_ref, o_ref):
        d1 = dinv_ref[:, :1]
        h = a_ref[0] + a_ref[1] - y_ref[0]
        o_ref[...] = h * d1 + b_ref[...]

    return pl.pallas_call(
        body,
        grid=(NP // BM,),
        in_specs=[
            pl.BlockSpec((NC, BM, C), lambda i: (0, i, 0)),
            pl.BlockSpec((1, BM, C), lambda i: (0, i, 0)),
            pl.BlockSpec((BM, 128), lambda i: (i, 0)),
            pl.BlockSpec((1, C), lambda i: (0, 0)),
        ],
        out_specs=pl.BlockSpec((BM, C), lambda i: (i, 0)),
        out_shape=jax.ShapeDtypeStruct((NP, C), jnp.float32),
    )(agg4, y4, dinv16, b4)


def _padc(v, n):
    return jnp.pad(v, (0, n - v.shape[0]))[None, :]


def kernel(x, edge_index, W1, b1, W2, b2, W3, b3, W4, b4, g1, be1, g2, be2, g3, be3):
    src = edge_index[0]
    dst = edge_index[1]
    x = jnp.pad(x, ((0, NP - NN), (0, 0)))

    W1p = jnp.pad(W1, ((0, 0), (0, 512 - 500)))
    W2p = jnp.pad(W2, ((0, 512 - 500), (0, 1024 - 1000)))
    W3p = jnp.pad(W3, ((0, 1024 - 1000), (0, 512 - 500)))
    W4p = jnp.pad(W4, ((0, 512 - 500), (0, 128 - 50)))
    b1p, g1p, be1p = _padc(b1, 512), _padc(g1, 512), _padc(be1, 512)
    b2p, g2p, be2p = _padc(b2, 1024), _padc(g2, 1024), _padc(be2, 1024)
    b3p, g3p, be3p = _padc(b3, 512), _padc(g3, 512), _padc(be3, 512)
    b4p = _padc(b4, 128)

    zinit = jnp.zeros((NP, 128), jnp.float32)
    ones_rows = jnp.ones((CH, 128), jnp.float32)

    deg2 = _degree(dst, zinit, ones_rows)
    y1, dinv16 = _mm_first(x, W1p, deg2)
    a1 = _make_agg(4, 128)(y1, src, dst)
    y2 = _mm_mid(a1, dinv16, b1p, g1p, be1p, W2p, 128)
    a2 = _make_agg(8, 128)(y2, src, dst)
    y3 = _mm_mid(a2, dinv16, b2p, g2p, be2p, W3p, 128)
    a3 = _make_agg(4, 128)(y3, src, dst)
    y4 = _mm_mid(a3, dinv16, b3p, g3p, be3p, W4p, 128)
    a4 = _make_agg(1, 128, True)(y4, src, dst)
    out = _final(a4, y4, dinv16, b4p)
    return out[:NN, :50]


# speedup vs baseline: 1.5965x; 1.0275x over previous
import functools
import math

import jax
import jax.numpy as jnp
from jax import lax
from jax.experimental import pallas as pl
from jax.experimental.pallas import tpu as pltpu
from jax.experimental.pallas import tpu_sc as plsc

NN = 10000
NP = 10240
EE = 320000
EPS = 1e-5

NC = 2
NS = 16
RPT = NP // NS
EPT = EE // NS
CH = 80
BM = 512


def _sc_mesh():
    return plsc.VectorSubcoreMesh(core_axis_name="c", subcore_axis_name="s")


def _degree(dst, zeros_init, ones_rows):
    IS = 8
    NITERD = EE // NC // NS // CH
    NMAIND = (NITERD // IS) * IS

    @functools.partial(
        pl.kernel,
        out_type=jax.ShapeDtypeStruct((NC, NP, 128), jnp.float32),
        mesh=_sc_mesh(),
        scratch_types=[
            pltpu.VMEM_SHARED((NP, 128), jnp.float32),
            pltpu.VMEM((IS, CH), jnp.int32),
            pltpu.VMEM((CH, 128), jnp.float32),
            pltpu.SemaphoreType.DMA,
            pltpu.SemaphoreType.DMA,
        ],
    )
    def deg_kernel(dst_hbm, zinit_hbm, ones_hbm, out_hbm, acc, didx, ones_v,
                   isem, ssem):
        c = lax.axis_index("c")
        s = lax.axis_index("s")
        pltpu.sync_copy(ones_hbm, ones_v)
        pltpu.sync_copy(zinit_hbm.at[pl.ds(s * RPT, RPT)],
                        acc.at[pl.ds(s * RPT, RPT)])
        plsc.subcore_barrier()
        ebase = c * (EE // NC) + s * (EE // NC // NS)

        def fetch_idx(j, sl):
            pltpu.async_copy(dst_hbm.at[pl.ds(ebase + j * CH, CH)],
                             didx.at[sl], isem)

        def wait_idx(sl):
            pltpu.make_async_copy(dst_hbm.at[pl.ds(0, CH)], didx.at[sl],
                                  isem).wait()

        def wait_scatter():
            pltpu.make_async_copy(ones_v, acc.at[didx.at[0]], ssem).wait()

        for j0 in range(5):
            fetch_idx(j0, j0)

        def step(j, ib):
            wait_idx(ib)
            pltpu.async_copy(ones_v, acc.at[didx.at[ib]], ssem, add=True)

            @pl.when(j >= 3)
            def _():
                wait_scatter()

            @pl.when(j + 5 < NITERD)
            def _():
                fetch_idx(j + 5, (ib + 5) % IS)

        def group(g, _):
            for b in range(IS):
                step(g * IS + b, b)
            return 0

        lax.fori_loop(0, NMAIND // IS, group, 0)
        for j in range(NMAIND, NITERD):
            step(j, j % IS)
        for _ in range(3):
            wait_scatter()
        plsc.subcore_barrier()
        pltpu.sync_copy(acc.at[pl.ds(s * RPT, RPT)],
                        out_hbm.at[c, pl.ds(s * RPT, RPT)])

    return deg_kernel(dst, zeros_init, ones_rows)


NITER = EPT // CH
NBUF = 2


@functools.lru_cache(maxsize=None)
def _make_agg(NB, C, edge_split=False):
    NBC = 1 if edge_split else NB // NC
    RS = 4
    IS = 8
    NITERL = (EPT // NC if edge_split else EPT) // CH
    NMAIN = (NITERL // IS) * IS
    NOUT = NC if edge_split else NB

    @functools.partial(
        pl.kernel,
        out_type=jax.ShapeDtypeStruct((NOUT, NP, C), jnp.float32),
        mesh=_sc_mesh(),
        scratch_types=[
            pltpu.VMEM_SHARED((NP, C), jnp.float32),
            pltpu.VMEM((IS, CH), jnp.int32),
            pltpu.VMEM((IS, CH), jnp.int32),
            pltpu.VMEM((RS, CH, C), jnp.float32),
            pltpu.SemaphoreType.DMA,
            pltpu.SemaphoreType.DMA,
            pltpu.SemaphoreType.DMA,
        ],
    )
    def agg_kernel(y_hbm, src_hbm, dst_hbm, out_hbm, acc, sidx, didx, rows,
                   isem, gsem, ssem):
        c = lax.axis_index("c")
        s = lax.axis_index("s")
        if edge_split:
            ebase = c * (EE // NC) + s * (EPT // NC)
        else:
            ebase = s * EPT

        def fetch_idx(j, sl):
            pltpu.async_copy(src_hbm.at[pl.ds(ebase + j * CH, CH)],
                             sidx.at[sl], isem)
            pltpu.async_copy(dst_hbm.at[pl.ds(ebase + j * CH, CH)],
                             didx.at[sl], isem)

        def wait_idx(sl):
            pltpu.make_async_copy(src_hbm.at[pl.ds(0, CH)], sidx.at[sl],
                                  isem).wait()
            pltpu.make_async_copy(dst_hbm.at[pl.ds(0, CH)], didx.at[sl],
                                  isem).wait()

        def wait_gather(y_fb, rb):
            pltpu.make_async_copy(y_fb.at[sidx.at[0]], rows.at[rb],
                                  gsem).wait()

        def wait_scatter(rb):
            pltpu.make_async_copy(rows.at[rb], acc.at[didx.at[0]],
                                  ssem).wait()

        for bi in range(NBC):
            fb = 0 if edge_split else c * NBC + bi
            ob = c if edge_split else fb
            y_fb = y_hbm.at[fb]
            pltpu.sync_copy(y_hbm.at[fb, pl.ds(s * RPT, RPT)],
                            acc.at[pl.ds(s * RPT, RPT)])
            plsc.subcore_barrier()

            for j0 in range(5):
                fetch_idx(j0, j0)
            wait_idx(0)
            pltpu.async_copy(y_fb.at[sidx.at[0]], rows.at[0], gsem)
            wait_idx(1)
            pltpu.async_copy(y_fb.at[sidx.at[1]], rows.at[1], gsem)

            def step(j, ib, rb):
                wait_gather(y_fb, rb)
                pltpu.async_copy(rows.at[rb], acc.at[didx.at[ib]], ssem,
                                 add=True)

                @pl.when(j >= 2)
                def _():
                    wait_scatter((rb + 2) % RS)

                @pl.when(j + 2 < NITERL)
                def _():
                    wait_idx((ib + 2) % IS)
                    pltpu.async_copy(y_fb.at[sidx.at[(ib + 2) % IS]],
                                     rows.at[(rb + 2) % RS], gsem)

                @pl.when(j + 5 < NITERL)
                def _():
                    fetch_idx(j + 5, (ib + 5) % IS)

            def group(g, _):
                for b in range(IS):
                    step(g * IS + b, b, b % RS)
                return 0

            lax.fori_loop(0, NMAIN // IS, group, 0)
            for j in range(NMAIN, NITERL):
                step(j, j % IS, j % RS)
            for j in range(NITERL - 2, NITERL):
                wait_scatter(j % RS)
            plsc.subcore_barrier()
            pltpu.sync_copy(acc.at[pl.ds(s * RPT, RPT)],
                            out_hbm.at[ob, pl.ds(s * RPT, RPT)])
            if bi + 1 < NBC:
                plsc.subcore_barrier()

    return agg_kernel


def _mm_plain(x, W):
    F = W.shape[1]

    def body(x_ref, w_ref, y_ref):
        y = jnp.dot(x_ref[...], w_ref[...], preferred_element_type=jnp.float32)
        for fbl in range(F // 128):
            y_ref[fbl] = y[:, fbl * 128:(fbl + 1) * 128]

    return pl.pallas_call(
        body,
        grid=(NP // BM,),
        in_specs=[
            pl.BlockSpec((BM, x.shape[1]), lambda i: (i, 0)),
            pl.BlockSpec(W.shape, lambda i: (0, 0)),
        ],
        out_specs=[pl.BlockSpec((F // 128, BM, 128), lambda i: (0, i, 0))],
        out_shape=[jax.ShapeDtypeStruct((F // 128, NP, 128), jnp.float32)],
    )(x, W)[0]


def _scale_first(xw, deg2):
    NBf = xw.shape[0]

    def body(xw_ref, deg_ref, y_ref, dinv_ref):
        deg = deg_ref[0] + deg_ref[1] + 1.0
        dinv = lax.rsqrt(deg)
        dinv_ref[...] = dinv
        d1 = dinv[:, :1]
        for fbl in range(NBf):
            y_ref[fbl] = xw_ref[fbl] * d1

    return pl.pallas_call(
        body,
        grid=(NP // BM,),
        in_specs=[
            pl.BlockSpec((NBf, BM, 128), lambda i: (0, i, 0)),
            pl.BlockSpec((NC, BM, 128), lambda i: (0, i, 0)),
        ],
        out_specs=[
            pl.BlockSpec((NBf, BM, 128), lambda i: (0, i, 0)),
            pl.BlockSpec((BM, 128), lambda i: (i, 0)),
        ],
        out_shape=[
            jax.ShapeDtypeStruct((NBf, NP, 128), jnp.float32),
            jax.ShapeDtypeStruct((NP, 128), jnp.float32),
        ],
    )(xw, deg2)


def _mm_mid(agg, dinv16, b, g, be, W, C_out):
    NBin, _, C_in = agg.shape
    Fout = W.shape[1]
    NBo = Fout // C_out
    bnscale = 1.0 / math.sqrt(1.0 + EPS)

    def body(a_ref, dinv_ref, b_ref, g_ref, be_ref, w_ref, y_ref):
        d1 = dinv_ref[:, :1]
        acc = jnp.zeros((BM, Fout), jnp.float32)
        for kb in range(NBin):
            sl = slice(kb * C_in, (kb + 1) * C_in)
            h = a_ref[kb] * d1 + b_ref[0:1, sl]
            h = jnp.where(h > 0, h, jnp.exp(h) - 1.0)
            h = h * (g_ref[0:1, sl] * bnscale) + be_ref[0:1, sl]
            acc = acc + jnp.dot(h, w_ref[sl, :],
                                preferred_element_type=jnp.float32)
        y = acc * d1
        for fbl in range(NBo):
            y_ref[fbl] = y[:, fbl * C_out:(fbl + 1) * C_out]

    Fin = NBin * C_in
    return pl.pallas_call(
        body,
        grid=(NP // BM,),
        in_specs=[
            pl.BlockSpec((NBin, BM, C_in), lambda i: (0, i, 0)),
            pl.BlockSpec((BM, 128), lambda i: (i, 0)),
            pl.BlockSpec((1, Fin), lambda i: (0, 0)),
            pl.BlockSpec((1, Fin), lambda i: (0, 0)),
            pl.BlockSpec((1, Fin), lambda i: (0, 0)),
            pl.BlockSpec((Fin, Fout), lambda i: (0, 0)),
        ],
        out_specs=[pl.BlockSpec((NBo, BM, C_out), lambda i: (0, i, 0))],
        out_shape=[jax.ShapeDtypeStruct((NBo, NP, C_out), jnp.float32)],
    )(agg, dinv16, b, g, be, W)[0]


def _final(agg4, y4, dinv16, b4):
    C = agg4.shape[2]

    def body(a_ref, y_ref, dinv_ref, b_ref, o_ref):
        d1 = dinv_ref[:, :1]
        h = a_ref[0] + a_ref[1] - y_ref[0]
        o_ref[...] = h * d1 + b_ref[...]

    return pl.pallas_call(
        body,
        grid=(NP // BM,),
        in_specs=[
            pl.BlockSpec((NC, BM, C), lambda i: (0, i, 0)),
            pl.BlockSpec((1, BM, C), lambda i: (0, i, 0)),
            pl.BlockSpec((BM, 128), lambda i: (i, 0)),
            pl.BlockSpec((1, C), lambda i: (0, 0)),
        ],
        out_specs=pl.BlockSpec((BM, C), lambda i: (i, 0)),
        out_shape=jax.ShapeDtypeStruct((NP, C), jnp.float32),
    )(agg4, y4, dinv16, b4)


def _padc(v, n):
    return jnp.pad(v, (0, n - v.shape[0]))[None, :]


def kernel(x, edge_index, W1, b1, W2, b2, W3, b3, W4, b4, g1, be1, g2, be2, g3, be3):
    src = edge_index[0]
    dst = edge_index[1]
    x = jnp.pad(x, ((0, NP - NN), (0, 0)))

    W1p = jnp.pad(W1, ((0, 0), (0, 512 - 500)))
    W2p = jnp.pad(W2, ((0, 512 - 500), (0, 1024 - 1000)))
    W3p = jnp.pad(W3, ((0, 1024 - 1000), (0, 512 - 500)))
    W4p = jnp.pad(W4, ((0, 512 - 500), (0, 128 - 50)))
    b1p, g1p, be1p = _padc(b1, 512), _padc(g1, 512), _padc(be1, 512)
    b2p, g2p, be2p = _padc(b2, 1024), _padc(g2, 1024), _padc(be2, 1024)
    b3p, g3p, be3p = _padc(b3, 512), _padc(g3, 512), _padc(be3, 512)
    b4p = _padc(b4, 128)

    zinit = jnp.zeros((NP, 128), jnp.float32)
    ones_rows = jnp.ones((CH, 128), jnp.float32)

    deg2 = _degree(dst, zinit, ones_rows)
    xw1 = _mm_plain(x, W1p)
    y1, dinv16 = _scale_first(xw1, deg2)
    a1 = _make_agg(4, 128)(y1, src, dst)
    y2 = _mm_mid(a1, dinv16, b1p, g1p, be1p, W2p, 128)
    a2 = _make_agg(8, 128)(y2, src, dst)
    y3 = _mm_mid(a2, dinv16, b2p, g2p, be2p, W3p, 128)
    a3 = _make_agg(4, 128)(y3, src, dst)
    y4 = _mm_mid(a3, dinv16, b3p, g3p, be3p, W4p, 128)
    a4 = _make_agg(1, 128, True)(y4, src, dst)
    out = _final(a4, y4, dinv16, b4p)
    return out[:NN, :50]


# speedup vs baseline: 1.5972x; 1.0004x over previous
import functools
import math

import jax
import jax.numpy as jnp
from jax import lax
from jax.experimental import pallas as pl
from jax.experimental.pallas import tpu as pltpu
from jax.experimental.pallas import tpu_sc as plsc

NN = 10000
NP = 10240
EE = 320000
EPS = 1e-5

NC = 2
NS = 16
RPT = NP // NS
EPT = EE // NS
CH = 80
BM = 512


def _sc_mesh():
    return plsc.VectorSubcoreMesh(core_axis_name="c", subcore_axis_name="s")


def _degree(dst, zeros_init, ones_rows):
    IS = 8
    NITERD = EE // NC // NS // CH
    NMAIND = (NITERD // IS) * IS

    @functools.partial(
        pl.kernel,
        out_type=jax.ShapeDtypeStruct((NC, NP, 128), jnp.float32),
        mesh=_sc_mesh(),
        scratch_types=[
            pltpu.VMEM_SHARED((NP, 128), jnp.float32),
            pltpu.VMEM((IS, CH), jnp.int32),
            pltpu.VMEM((CH, 128), jnp.float32),
            pltpu.SemaphoreType.DMA,
            pltpu.SemaphoreType.DMA,
        ],
    )
    def deg_kernel(dst_hbm, zinit_hbm, ones_hbm, out_hbm, acc, didx, ones_v,
                   isem, ssem):
        c = lax.axis_index("c")
        s = lax.axis_index("s")
        pltpu.sync_copy(ones_hbm, ones_v)
        pltpu.sync_copy(zinit_hbm.at[pl.ds(s * RPT, RPT)],
                        acc.at[pl.ds(s * RPT, RPT)])
        plsc.subcore_barrier()
        ebase = c * (EE // NC) + s * (EE // NC // NS)

        def fetch_idx(j, sl):
            pltpu.async_copy(dst_hbm.at[pl.ds(ebase + j * CH, CH)],
                             didx.at[sl], isem)

        def wait_idx(sl):
            pltpu.make_async_copy(dst_hbm.at[pl.ds(0, CH)], didx.at[sl],
                                  isem).wait()

        def wait_scatter():
            pltpu.make_async_copy(ones_v, acc.at[didx.at[0]], ssem).wait()

        for j0 in range(5):
            fetch_idx(j0, j0)

        def step(j, ib):
            wait_idx(ib)
            pltpu.async_copy(ones_v, acc.at[didx.at[ib]], ssem, add=True)

            @pl.when(j >= 3)
            def _():
                wait_scatter()

            @pl.when(j + 5 < NITERD)
            def _():
                fetch_idx(j + 5, (ib + 5) % IS)

        def group(g, _):
            for b in range(IS):
                step(g * IS + b, b)
            return 0

        lax.fori_loop(0, NMAIND // IS, group, 0)
        for j in range(NMAIND, NITERD):
            step(j, j % IS)
        for _ in range(3):
            wait_scatter()
        plsc.subcore_barrier()
        pltpu.sync_copy(acc.at[pl.ds(s * RPT, RPT)],
                        out_hbm.at[c, pl.ds(s * RPT, RPT)])

    return deg_kernel(dst, zeros_init, ones_rows)


NITER = EPT // CH


@functools.lru_cache(maxsize=None)
def _make_agg(NB, C, edge_split=False):
    NBC = 1 if edge_split else NB // NC
    RS = 4
    IS = 8
    NITERL = (EPT // NC if edge_split else EPT) // CH
    NMAIN = (NITERL // IS) * IS
    NOUT = NC if edge_split else NB

    @functools.partial(
        pl.kernel,
        out_type=jax.ShapeDtypeStruct((NOUT, NP, C), jnp.float32),
        mesh=_sc_mesh(),
        scratch_types=[
            pltpu.VMEM_SHARED((NP, C), jnp.float32),
            pltpu.VMEM((IS, CH), jnp.int32),
            pltpu.VMEM((IS, CH), jnp.int32),
            pltpu.VMEM((RS, CH, C), jnp.float32),
            pltpu.SemaphoreType.DMA,
            pltpu.SemaphoreType.DMA,
            pltpu.SemaphoreType.DMA,
        ],
    )
    def agg_kernel(y_hbm, src_hbm, dst_hbm, out_hbm, acc, sidx, didx, rows,
                   isem, gsem, ssem):
        c = lax.axis_index("c")
        s = lax.axis_index("s")
        if edge_split:
            ebase = c * (EE // NC) + s * (EPT // NC)
        else:
            ebase = s * EPT

        def fetch_idx(j, sl):
            pltpu.async_copy(src_hbm.at[pl.ds(ebase + j * CH, CH)],
                             sidx.at[sl], isem)
            pltpu.async_copy(dst_hbm.at[pl.ds(ebase + j * CH, CH)],
                             didx.at[sl], isem)

        def wait_idx(sl):
            pltpu.make_async_copy(src_hbm.at[pl.ds(0, CH)], sidx.at[sl],
                                  isem).wait()
            pltpu.make_async_copy(dst_hbm.at[pl.ds(0, CH)], didx.at[sl],
                                  isem).wait()

        def wait_gather(y_fb, rb):
            pltpu.make_async_copy(y_fb.at[sidx.at[0]], rows.at[rb],
                                  gsem).wait()

        def wait_scatter(rb):
            pltpu.make_async_copy(rows.at[rb], acc.at[didx.at[0]],
                                  ssem).wait()

        for bi in range(NBC):
            fb = 0 if edge_split else c * NBC + bi
            ob = c if edge_split else fb
            y_fb = y_hbm.at[fb]
            pltpu.sync_copy(y_hbm.at[fb, pl.ds(s * RPT, RPT)],
                            acc.at[pl.ds(s * RPT, RPT)])
            plsc.subcore_barrier()

            for j0 in range(5):
                fetch_idx(j0, j0)
            wait_idx(0)
            pltpu.async_copy(y_fb.at[sidx.at[0]], rows.at[0], gsem)
            wait_idx(1)
            pltpu.async_copy(y_fb.at[sidx.at[1]], rows.at[1], gsem)

            def step(j, ib, rb):
                wait_gather(y_fb, rb)
                pltpu.async_copy(rows.at[rb], acc.at[didx.at[ib]], ssem,
                                 add=True)

                @pl.when(j >= 2)
                def _():
                    wait_scatter((rb + 2) % RS)

                @pl.when(j + 2 < NITERL)
                def _():
                    wait_idx((ib + 2) % IS)
                    pltpu.async_copy(y_fb.at[sidx.at[(ib + 2) % IS]],
                                     rows.at[(rb + 2) % RS], gsem)

                @pl.when(j + 5 < NITERL)
                def _():
                    fetch_idx(j + 5, (ib + 5) % IS)

            def group(g, _):
                for b in range(IS):
                    step(g * IS + b, b, b % RS)
                return 0

            lax.fori_loop(0, NMAIN // IS, group, 0)
            for j in range(NMAIN, NITERL):
                step(j, j % IS, j % RS)
            for j in range(NITERL - 2, NITERL):
                wait_scatter(j % RS)
            plsc.subcore_barrier()
            pltpu.sync_copy(acc.at[pl.ds(s * RPT, RPT)],
                            out_hbm.at[ob, pl.ds(s * RPT, RPT)])
            if bi + 1 < NBC:
                plsc.subcore_barrier()

    return agg_kernel


def _mm_plain(x, W):
    F = W.shape[1]

    def body(x_ref, w_ref, y_ref):
        y = jnp.dot(x_ref[...], w_ref[...], preferred_element_type=jnp.float32)
        for fbl in range(F // 128):
            y_ref[fbl] = y[:, fbl * 128:(fbl + 1) * 128]

    return pl.pallas_call(
        body,
        grid=(NP // BM,),
        in_specs=[
            pl.BlockSpec((BM, x.shape[1]), lambda i: (i, 0)),
            pl.BlockSpec(W.shape, lambda i: (0, 0)),
        ],
        out_specs=[pl.BlockSpec((F // 128, BM, 128), lambda i: (0, i, 0))],
        out_shape=[jax.ShapeDtypeStruct((F // 128, NP, 128), jnp.float32)],
    )(x, W)[0]


def _scale_first(xw, deg2):
    NBf = xw.shape[0]

    def body(xw_ref, deg_ref, y_ref, dinv_ref):
        deg = deg_ref[0] + deg_ref[1] + 1.0
        dinv = lax.rsqrt(deg)
        dinv_ref[...] = dinv
        d1 = dinv[:, :1]
        for fbl in range(NBf):
            y_ref[fbl] = xw_ref[fbl] * d1

    return pl.pallas_call(
        body,
        grid=(NP // BM,),
        in_specs=[
            pl.BlockSpec((NBf, BM, 128), lambda i: (0, i, 0)),
            pl.BlockSpec((NC, BM, 128), lambda i: (0, i, 0)),
        ],
        out_specs=[
            pl.BlockSpec((NBf, BM, 128), lambda i: (0, i, 0)),
            pl.BlockSpec((BM, 128), lambda i: (i, 0)),
        ],
        out_shape=[
            jax.ShapeDtypeStruct((NBf, NP, 128), jnp.float32),
            jax.ShapeDtypeStruct((NP, 128), jnp.float32),
        ],
    )(xw, deg2)


def _mm_mid(agg, dinv16, b, g, be, W, C_out):
    NBin, _, C_in = agg.shape
    Fout = W.shape[1]
    NBo = Fout // C_out
    bnscale = 1.0 / math.sqrt(1.0 + EPS)

    def body(a_ref, dinv_ref, b_ref, g_ref, be_ref, w_ref, y_ref):
        d1 = dinv_ref[:, :1]
        acc = jnp.zeros((BM, Fout), jnp.float32)
        for kb in range(NBin):
            sl = slice(kb * C_in, (kb + 1) * C_in)
            h = a_ref[kb] * d1 + b_ref[0:1, sl]
            h = jnp.where(h > 0, h, jnp.exp(h) - 1.0)
            h = h * (g_ref[0:1, sl] * bnscale) + be_ref[0:1, sl]
            acc = acc + jnp.dot(h, w_ref[sl, :],
                                preferred_element_type=jnp.float32)
        y = acc * d1
        for fbl in range(NBo):
            y_ref[fbl] = y[:, fbl * C_out:(fbl + 1) * C_out]

    Fin = NBin * C_in
    return pl.pallas_call(
        body,
        grid=(NP // BM,),
        in_specs=[
            pl.BlockSpec((NBin, BM, C_in), lambda i: (0, i, 0)),
            pl.BlockSpec((BM, 128), lambda i: (i, 0)),
            pl.BlockSpec((1, Fin), lambda i: (0, 0)),
            pl.BlockSpec((1, Fin), lambda i: (0, 0)),
            pl.BlockSpec((1, Fin), lambda i: (0, 0)),
            pl.BlockSpec((Fin, Fout), lambda i: (0, 0)),
        ],
        out_specs=[pl.BlockSpec((NBo, BM, C_out), lambda i: (0, i, 0))],
        out_shape=[jax.ShapeDtypeStruct((NBo, NP, C_out), jnp.float32)],
    )(agg, dinv16, b, g, be, W)[0]


def _final(agg4, y4, dinv16, b4):
    C = agg4.shape[2]

    def body(a_ref, y_ref, dinv_ref, b_ref, o_ref):
        d1 = dinv_ref[:, :1]
        h = a_ref[0] + a_ref[1] - y_ref[0]
        o_ref[...] = h * d1 + b_ref[...]

    return pl.pallas_call(
        body,
        grid=(NP // BM,),
        in_specs=[
            pl.BlockSpec((NC, BM, C), lambda i: (0, i, 0)),
            pl.BlockSpec((1, BM, C), lambda i: (0, i, 0)),
            pl.BlockSpec((BM, 128), lambda i: (i, 0)),
            pl.BlockSpec((1, C), lambda i: (0, 0)),
        ],
        out_specs=pl.BlockSpec((BM, C), lambda i: (i, 0)),
        out_shape=jax.ShapeDtypeStruct((NP, C), jnp.float32),
    )(agg4, y4, dinv16, b4)


def _padc(v, n):
    return jnp.pad(v, (0, n - v.shape[0]))[None, :]


def kernel(x, edge_index, W1, b1, W2, b2, W3, b3, W4, b4, g1, be1, g2, be2, g3, be3):
    src = edge_index[0]
    dst = edge_index[1]
    x = jnp.pad(x, ((0, NP - NN), (0, 0)))

    W1p = jnp.pad(W1, ((0, 0), (0, 512 - 500)))
    W2p = jnp.pad(W2, ((0, 512 - 500), (0, 1024 - 1000)))
    W3p = jnp.pad(W3, ((0, 1024 - 1000), (0, 512 - 500)))
    W4p = jnp.pad(W4, ((0, 512 - 500), (0, 128 - 50)))
    b1p, g1p, be1p = _padc(b1, 512), _padc(g1, 512), _padc(be1, 512)
    b2p, g2p, be2p = _padc(b2, 1024), _padc(g2, 1024), _padc(be2, 1024)
    b3p, g3p, be3p = _padc(b3, 512), _padc(g3, 512), _padc(be3, 512)
    b4p = _padc(b4, 128)

    zinit = jnp.zeros((NP, 128), jnp.float32)
    ones_rows = jnp.ones((CH, 128), jnp.float32)

    deg2 = _degree(dst, zinit, ones_rows)
    xw1 = _mm_plain(x, W1p)
    y1, dinv16 = _scale_first(xw1, deg2)
    a1 = _make_agg(4, 128)(y1, src, dst)
    y2 = _mm_mid(a1, dinv16, b1p, g1p, be1p, W2p, 128)
    a2 = _make_agg(8, 128)(y2, src, dst)
    y3 = _mm_mid(a2, dinv16, b2p, g2p, be2p, W3p, 128)
    a3 = _make_agg(4, 128)(y3, src, dst)
    y4 = _mm_mid(a3, dinv16, b3p, g3p, be3p, W4p, 128)
    a4 = _make_agg(1, 128, True)(y4, src, dst)
    out = _final(a4, y4, dinv16, b4p)
    return out[:NN, :50]
